# baseline (speedup 1.0000x reference)
import functools

import jax
import jax.numpy as jnp
from jax import lax
from jax.experimental import pallas as pl
from jax.experimental.pallas import tpu as pltpu
from jax.experimental.pallas import tpu_sc as plsc

B = 16384
EMBED = 16
NC, NS = 2, 16
NW = NC * NS
RPW = B // NW
CHUNK = 128
NCH = RPW // CHUNK
RB = 2048
NBLK = B // RB


def _sc_gather(user_table, item_table, uidx, pidx, nidx):
    mesh = plsc.VectorSubcoreMesh(core_axis_name="c", subcore_axis_name="s")
    out = jax.ShapeDtypeStruct((B, EMBED), jnp.float32)

    @functools.partial(
        pl.kernel,
        out_type=(out, out, out),
        mesh=mesh,
        compiler_params=pltpu.CompilerParams(use_tc_tiling_on_sc=False),
        scratch_types=[
            pltpu.VMEM((NCH, CHUNK), jnp.int32),
            pltpu.VMEM((NCH, CHUNK), jnp.int32),
            pltpu.VMEM((NCH, CHUNK), jnp.int32),
            pltpu.VMEM((RPW, EMBED), jnp.float32),
            pltpu.VMEM((RPW, EMBED), jnp.float32),
            pltpu.VMEM((RPW, EMBED), jnp.float32),
            pltpu.SemaphoreType.DMA,
        ],
    )
    def gather(ut, it, ui, pi, ni, uo, po, no,
               ui_v, pi_v, ni_v, ur_v, pr_v, nr_v, sem):
        wid = lax.axis_index("s") * NC + lax.axis_index("c")
        base = wid * RPW
        for src, iv in ((ui, ui_v), (pi, pi_v), (ni, ni_v)):
            for j in range(NCH):
                pltpu.sync_copy(src.at[pl.ds(base + j * CHUNK, CHUNK)], iv.at[j])
        copies = []
        for tab, iv, rv in ((ut, ui_v, ur_v), (it, pi_v, pr_v), (it, ni_v, nr_v)):
            for j in range(NCH):
                copies.append(pltpu.async_copy(
                    tab.at[iv.at[j]], rv.at[pl.ds(j * CHUNK, CHUNK)], sem))
        for c in copies:
            c.wait()
        pltpu.sync_copy(ur_v, uo.at[pl.ds(base, RPW)])
        pltpu.sync_copy(pr_v, po.at[pl.ds(base, RPW)])
        pltpu.sync_copy(nr_v, no.at[pl.ds(base, RPW)])

    return gather(user_table, item_table, uidx, pidx, nidx)


def _mlp_body(ue_ref, pe_ref, ne_ref,
              uw0, ub0, uw1, ub1, uw2, ub2, uw3, ub3,
              iw0, ib0, iw1, ib1, iw2, ib2, iw3, ib3,
              dw, db, out_ref):
    def mm(x, W):
        return jnp.dot(x, W, preferred_element_type=jnp.float32,
                       precision=lax.Precision.HIGHEST)

    u = ue_ref[...]
    for W, b in ((uw0, ub0), (uw1, ub1), (uw2, ub2), (uw3, ub3)):
        u = jnp.maximum(mm(u, W[...]) + b[...], 0.0)
    p = pe_ref[...]
    n = ne_ref[...]
    for W, b in ((iw0, ib0), (iw1, ib1), (iw2, ib2), (iw3, ib3)):
        Wv, bv = W[...], b[...]
        p = jnp.maximum(mm(p, Wv) + bv, 0.0)
        n = jnp.maximum(mm(n, Wv) + bv, 0.0)
    w = dw[...]
    bias = db[...]
    pv = jax.nn.sigmoid(u * p)
    nv = jax.nn.sigmoid(u * n)
    pos_l = jnp.sum(pv * w, axis=1, keepdims=True) + bias
    neg_l = jnp.sum(nv * w, axis=1, keepdims=True) + bias
    out_ref[...] = jnp.concatenate([pos_l, neg_l], axis=1)


def _tc_mlp(ue, pe, ne, weights):
    def wspec(w):
        return pl.BlockSpec(w.shape, lambda i: (0, 0))

    in_specs = [
        pl.BlockSpec((RB, EMBED), lambda i: (i, 0)),
        pl.BlockSpec((RB, EMBED), lambda i: (i, 0)),
        pl.BlockSpec((RB, EMBED), lambda i: (i, 0)),
    ] + [wspec(w) for w in weights]

    return pl.pallas_call(
        _mlp_body,
        grid=(NBLK,),
        in_specs=in_specs,
        out_specs=pl.BlockSpec((RB, 2), lambda i: (i, 0)),
        out_shape=jax.ShapeDtypeStruct((B, 2), jnp.float32),
    )(ue, pe, ne, *weights)


def kernel(user, pos, neg, user_table, item_table,
           uW0, ub0, uW1, ub1, uW2, ub2, uW3, ub3,
           iW0, ib0, iW1, ib1, iW2, ib2, iW3, ib3,
           dW, db):
    uidx = user.reshape(-1).astype(jnp.int32)
    pidx = pos.reshape(-1).astype(jnp.int32)
    nidx = neg.reshape(-1).astype(jnp.int32)

    ue, pe, ne = _sc_gather(user_table, item_table, uidx, pidx, nidx)

    weights = (
        uW0, ub0.reshape(1, -1), uW1, ub1.reshape(1, -1),
        uW2, ub2.reshape(1, -1), uW3, ub3.reshape(1, -1),
        iW0, ib0.reshape(1, -1), iW1, ib1.reshape(1, -1),
        iW2, ib2.reshape(1, -1), iW3, ib3.reshape(1, -1),
        dW.reshape(1, -1), db.reshape(1, 1),
    )
    return _tc_mlp(ue, pe, ne, weights)
